# Initial kernel scaffold; baseline (speedup 1.0000x reference)
import jax
import jax.numpy as jnp
from jax.experimental import pallas as pl


def kernel(node_feat, edge_index, edge_feat, W1, b1, g1, be1, W2, b2, g2, be2, We, bee, Wg, bg, Wc, bc):
    raise NotImplementedError("write your pallas kernel here")


# speedup vs baseline: 2.7041x; 2.7041x over previous
import functools

import jax
import jax.numpy as jnp
from jax import lax
from jax.experimental import pallas as pl
from jax.experimental.pallas import tpu as pltpu
from jax.experimental.pallas import tpu_sc as plsc

N = 10000
E = 160000
D = 256
H = 512
C = 10

NS = 16
NCORES = 2
LB = 128
EPT = E // NS
CHUNK = 80
NCHUNK = EPT // CHUNK
NPAD = 10240
RPT = NPAD // NS


def _deg_body(src_hbm, dst_hbm, ones_hbm, zeros_hbm, deg_hbm, idx_v, ones_v, acc_sh):
    c = lax.axis_index("c")
    s = lax.axis_index("s")
    pltpu.sync_copy(ones_hbm, ones_v)
    pltpu.sync_copy(zeros_hbm, acc_sh.at[pl.ds(s * RPT, RPT), :])
    plsc.subcore_barrier()

    def chunk_src(k, _):
        off = s * EPT + k * CHUNK
        pltpu.sync_copy(src_hbm.at[pl.ds(off, CHUNK)], idx_v)
        pltpu.sync_copy(ones_v, acc_sh.at[idx_v], add=True)
        return ()

    def chunk_dst(k, _):
        off = s * EPT + k * CHUNK
        pltpu.sync_copy(dst_hbm.at[pl.ds(off, CHUNK)], idx_v)
        pltpu.sync_copy(ones_v, acc_sh.at[idx_v], add=True)
        return ()

    @pl.when(c == 0)
    def _():
        lax.fori_loop(0, NCHUNK, chunk_src, ())

    @pl.when(c == 1)
    def _():
        lax.fori_loop(0, NCHUNK, chunk_dst, ())

    plsc.subcore_barrier()
    pltpu.sync_copy(acc_sh.at[pl.ds(s * RPT, RPT), :],
                    deg_hbm.at[c, pl.ds(s * RPT, RPT), :])


def _deg_call(src, dst):
    ones = jnp.ones((CHUNK, LB), jnp.float32)
    zeros = jnp.zeros((RPT, LB), jnp.float32)
    mesh = plsc.VectorSubcoreMesh(core_axis_name="c", subcore_axis_name="s")
    fn = pl.kernel(
        _deg_body,
        out_type=jax.ShapeDtypeStruct((2, NPAD, LB), jnp.float32),
        mesh=mesh,
        scratch_types=[
            pltpu.VMEM((CHUNK,), jnp.int32),
            pltpu.VMEM((CHUNK, LB), jnp.float32),
            pltpu.VMEM_SHARED((NPAD, LB), jnp.float32),
        ],
    )
    return fn(src, dst, ones, zeros)


def _agg_body(nblk, h_hbm, src_hbm, dst_hbm, zeros_hbm, agg_hbm,
              sidx_v, didx_v, rows_v, acc_sh, sem):
    c = lax.axis_index("c")
    s = lax.axis_index("s")
    for r in range(nblk // NCORES):
        pltpu.sync_copy(zeros_hbm, acc_sh.at[pl.ds(s * RPT, RPT), :])
        plsc.subcore_barrier()

        for cc in range(NCORES):
            cb = cc + NCORES * r

            def chunk(k, _, cb=cb):
                off = s * EPT + k * CHUNK
                pltpu.sync_copy(src_hbm.at[pl.ds(off, CHUNK)], sidx_v)
                pltpu.sync_copy(dst_hbm.at[pl.ds(off, CHUNK)], didx_v)
                pltpu.async_copy(h_hbm.at[cb].at[sidx_v], rows_v, sem).wait()
                pltpu.sync_copy(rows_v, acc_sh.at[didx_v], add=True)
                return ()

            @pl.when(c == cc)
            def _():
                lax.fori_loop(0, NCHUNK, chunk, ())

        plsc.subcore_barrier()
        for cc in range(NCORES):
            cb = cc + NCORES * r

            @pl.when(c == cc)
            def _(cb=cb):
                pltpu.sync_copy(acc_sh.at[pl.ds(s * RPT, RPT), :],
                                agg_hbm.at[cb, pl.ds(s * RPT, RPT), :])

        if r + 1 < nblk // NCORES:
            plsc.subcore_barrier()


def _agg_call(h_blk, src, dst):
    nblk = h_blk.shape[0]
    zeros = jnp.zeros((RPT, LB), jnp.float32)
    mesh = plsc.VectorSubcoreMesh(core_axis_name="c", subcore_axis_name="s")
    fn = pl.kernel(
        functools.partial(_agg_body, nblk),
        out_type=jax.ShapeDtypeStruct((nblk, NPAD, LB), jnp.float32),
        mesh=mesh,
        scratch_types=[
            pltpu.VMEM((CHUNK,), jnp.int32),
            pltpu.VMEM((CHUNK,), jnp.int32),
            pltpu.VMEM((CHUNK, LB), jnp.float32),
            pltpu.VMEM_SHARED((NPAD, LB), jnp.float32),
            pltpu.SemaphoreType.DMA,
        ],
    )
    return fn(h_blk, src, dst, zeros)


NB = 8
RB = NPAD // NB


def _prescale_body(x_ref, deg_ref, xs_ref, rout_ref, rin_ref):
    rout = lax.rsqrt(jnp.maximum(deg_ref[0, :, 0:1], 1.0))
    rin = lax.rsqrt(jnp.maximum(deg_ref[1, :, 0:1], 1.0))
    rout_ref[...] = rout
    rin_ref[...] = rin
    x = x_ref[...]
    for cb in range(D // LB):
        xs_ref[cb] = x[:, cb * LB:(cb + 1) * LB] * rout[:N]


def _prescale_call(x, deg):
    return pl.pallas_call(
        _prescale_body,
        out_shape=(
            jax.ShapeDtypeStruct((D // LB, N, LB), jnp.float32),
            jax.ShapeDtypeStruct((NPAD, 1), jnp.float32),
            jax.ShapeDtypeStruct((NPAD, 1), jnp.float32),
        ),
    )(x, deg)


def _mm_body(nblk, agg_ref, rin_ref, w_ref, z_ref, st_ref):
    i = pl.program_id(0)
    rin = rin_ref[...]
    z = jnp.zeros((RB, H), jnp.float32)
    for cb in range(nblk):
        a = agg_ref[cb] * rin
        z = z + jnp.dot(a, w_ref[cb * LB:(cb + 1) * LB, :],
                        preferred_element_type=jnp.float32)
    z_ref[...] = z
    cs = jnp.sum(z, axis=0, keepdims=True)
    css = jnp.sum(z * z, axis=0, keepdims=True)
    upd = jnp.concatenate([cs, css], axis=0)

    @pl.when(i == 0)
    def _():
        st_ref[...] = jnp.zeros((2, H), jnp.float32)

    st_ref[...] += upd


def _mm_call(agg, rin, w):
    nblk = agg.shape[0]
    return pl.pallas_call(
        functools.partial(_mm_body, nblk),
        grid=(NB,),
        in_specs=[
            pl.BlockSpec((nblk, RB, LB), lambda i: (0, i, 0)),
            pl.BlockSpec((RB, 1), lambda i: (i, 0)),
            pl.BlockSpec(w.shape, lambda i: (0, 0)),
        ],
        out_specs=(
            pl.BlockSpec((RB, H), lambda i: (i, 0)),
            pl.BlockSpec((2, H), lambda i: (0, 0)),
        ),
        out_shape=(
            jax.ShapeDtypeStruct((NPAD, H), jnp.float32),
            jax.ShapeDtypeStruct((2, H), jnp.float32),
        ),
    )(agg, rin, w)


def _norm_body(z_ref, st_ref, rout_ref, b_ref, g_ref, be_ref, h_ref):
    mu = st_ref[0:1, :] * (1.0 / N) + b_ref[...]
    ez2 = st_ref[1:2, :] * (1.0 / N)
    m0 = st_ref[0:1, :] * (1.0 / N)
    var = ez2 - m0 * m0
    zn = (z_ref[...] + b_ref[...] - mu) * lax.rsqrt(var + 1e-5) * g_ref[...] + be_ref[...]
    h = jnp.where(zn >= 0, zn, 0.01 * zn) * rout_ref[...]
    for co in range(H // LB):
        h_ref[co] = h[:, co * LB:(co + 1) * LB]


def _norm_call(z, st, rout, b, g, be):
    return pl.pallas_call(
        _norm_body,
        grid=(NB,),
        in_specs=[
            pl.BlockSpec((RB, H), lambda i: (i, 0)),
            pl.BlockSpec((2, H), lambda i: (0, 0)),
            pl.BlockSpec((RB, 1), lambda i: (i, 0)),
            pl.BlockSpec((1, H), lambda i: (0, 0)),
            pl.BlockSpec((1, H), lambda i: (0, 0)),
            pl.BlockSpec((1, H), lambda i: (0, 0)),
        ],
        out_specs=pl.BlockSpec((H // LB, RB, LB), lambda i: (0, i, 0)),
        out_shape=jax.ShapeDtypeStruct((H // LB, NPAD, LB), jnp.float32),
    )(z, st, rout, b.reshape(1, H), g.reshape(1, H), be.reshape(1, H))


def _readout_body(agg_ref, rin_ref, wg_ref, bg_ref, wc_ref, bc_ref, out_ref, acc):
    i = pl.program_id(0)
    rin = rin_ref[...]
    z = jnp.zeros((RB, H), jnp.float32)
    for cb in range(H // LB):
        a = agg_ref[cb] * rin
        z = z + jnp.dot(a, wg_ref[cb * LB:(cb + 1) * LB, :],
                        preferred_element_type=jnp.float32)
    z = z + bg_ref[...]
    hg = jnp.where(z >= 0, z, 0.01 * z)
    row = i * RB + lax.broadcasted_iota(jnp.int32, (RB, 1), 0)
    hg = jnp.where(row < N, hg, 0.0)

    @pl.when(i == 0)
    def _():
        acc[...] = jnp.zeros((1, H), jnp.float32)

    acc[...] += jnp.sum(hg, axis=0, keepdims=True)

    @pl.when(i == NB - 1)
    def _():
        m = acc[...] * (1.0 / N)
        out_ref[...] = jnp.dot(m, wc_ref[...],
                               preferred_element_type=jnp.float32) + bc_ref[...]


def _readout_call(agg, rin, wg, bg, wc, bc):
    return pl.pallas_call(
        _readout_body,
        grid=(NB,),
        in_specs=[
            pl.BlockSpec((H // LB, RB, LB), lambda i: (0, i, 0)),
            pl.BlockSpec((RB, 1), lambda i: (i, 0)),
            pl.BlockSpec((H, H), lambda i: (0, 0)),
            pl.BlockSpec((1, H), lambda i: (0, 0)),
            pl.BlockSpec((H, C), lambda i: (0, 0)),
            pl.BlockSpec((1, C), lambda i: (0, 0)),
        ],
        out_specs=pl.BlockSpec((1, C), lambda i: (0, 0)),
        out_shape=jax.ShapeDtypeStruct((1, C), jnp.float32),
        scratch_shapes=[pltpu.VMEM((1, H), jnp.float32)],
    )(agg, rin, wg, bg.reshape(1, H), wc, bc.reshape(1, C))


def kernel(node_feat, edge_index, edge_feat, W1, b1, g1, be1, W2, b2, g2, be2,
           We, bee, Wg, bg, Wc, bc):
    src = edge_index[0]
    dst = edge_index[1]
    deg = _deg_call(src, dst)
    xs, rout, rin = _prescale_call(node_feat, deg)
    agg1 = _agg_call(xs, src, dst)
    z1, st1 = _mm_call(agg1, rin, W1)
    h1 = _norm_call(z1, st1, rout, b1, g1, be1)
    agg2 = _agg_call(h1, src, dst)
    z2, st2 = _mm_call(agg2, rin, W2)
    h2 = _norm_call(z2, st2, rout, b2, g2, be2)
    agg3 = _agg_call(h2, src, dst)
    return _readout_call(agg3, rin, Wg[:H], bg, Wc, bc)


# speedup vs baseline: 6.2050x; 2.2947x over previous
import functools

import jax
import jax.numpy as jnp
from jax import lax
from jax.experimental import pallas as pl
from jax.experimental.pallas import tpu as pltpu
from jax.experimental.pallas import tpu_sc as plsc

N = 10000
E = 160000
D = 256
H = 512
C = 10

NS = 16
NCORES = 2
LB = 128
EPT = E // NS
CHUNK = 160
NPAD = 10240
RPT = NPAD // NS


TCH = E // CHUNK
NJ = (TCH + NS - 1) // NS


def _deg_body(src_hbm, dst_hbm, ones_hbm, zeros_hbm, deg_hbm,
              idx0, idx1, ones_v, acc_sh, semi0, semi1):
    c = lax.axis_index("c")
    s = lax.axis_index("s")
    idx = [idx0, idx1]
    semi = [semi0, semi1]
    pltpu.sync_copy(ones_hbm, ones_v)
    pltpu.sync_copy(zeros_hbm, acc_sh.at[pl.ds(s * RPT, RPT), :])
    plsc.subcore_barrier()

    for cc, e_hbm in ((0, src_hbm), (1, dst_hbm)):
        @pl.when(c == cc)
        def _(e_hbm=e_hbm):
            def idx_load(chunk, buf):
                return pltpu.async_copy(
                    e_hbm.at[pl.ds(chunk * CHUNK, CHUNK)], idx[buf], semi[buf])

            idx_load(s, 0)

            def step(j, par):
                cur = j * NS + s
                nxt = cur + NS

                @pl.when(nxt < TCH)
                def _():
                    idx_load(nxt, 1 - par)

                @pl.when(cur < TCH)
                def _():
                    pltpu.make_async_copy(
                        e_hbm.at[pl.ds(cur * CHUNK, CHUNK)], idx[par],
                        semi[par]).wait()
                    pltpu.sync_copy(ones_v, acc_sh.at[idx[par]], add=True)

            def pair(t, _):
                step(2 * t, 0)
                step(2 * t + 1, 1)
                return ()

            lax.fori_loop(0, NJ // 2, pair, ())
            step(NJ - 1, (NJ - 1) % 2)

    plsc.subcore_barrier()
    pltpu.sync_copy(acc_sh.at[pl.ds(s * RPT, RPT), :],
                    deg_hbm.at[c, pl.ds(s * RPT, RPT), :])


def _deg_call(src, dst):
    ones = jnp.ones((CHUNK, LB), jnp.float32)
    zeros = jnp.zeros((RPT, LB), jnp.float32)
    mesh = plsc.VectorSubcoreMesh(core_axis_name="c", subcore_axis_name="s")
    fn = pl.kernel(
        _deg_body,
        out_type=jax.ShapeDtypeStruct((2, NPAD, LB), jnp.float32),
        mesh=mesh,
        scratch_types=[
            pltpu.VMEM((CHUNK,), jnp.int32),
            pltpu.VMEM((CHUNK,), jnp.int32),
            pltpu.VMEM((CHUNK, LB), jnp.float32),
            pltpu.VMEM_SHARED((NPAD, LB), jnp.float32),
            pltpu.SemaphoreType.DMA,
            pltpu.SemaphoreType.DMA,
        ],
    )
    return fn(src, dst, ones, zeros)


def _agg_body(nblk, h_hbm, src_hbm, dst_hbm, zeros_hbm, agg_hbm,
              sidx0, sidx1, didx0, didx1, rows0, rows1, acc_sh,
              semg0, semg1, semi0, semi1):
    c = lax.axis_index("c")
    s = lax.axis_index("s")
    sidx = [sidx0, sidx1]
    didx = [didx0, didx1]
    rows = [rows0, rows1]
    semg = [semg0, semg1]
    semi = [semi0, semi1]

    for r in range(nblk // NCORES):
        pltpu.sync_copy(zeros_hbm, acc_sh.at[pl.ds(s * RPT, RPT), :])
        plsc.subcore_barrier()

        for cc in range(NCORES):
            cb = cc + NCORES * r

            @pl.when(c == cc)
            def _(cb=cb):
                def idx_load(chunk, buf):
                    pltpu.async_copy(
                        src_hbm.at[pl.ds(chunk * CHUNK, CHUNK)], sidx[buf],
                        semi[buf])
                    pltpu.async_copy(
                        dst_hbm.at[pl.ds(chunk * CHUNK, CHUNK)], didx[buf],
                        semi[buf])

                def idx_wait(chunk, buf):
                    pltpu.make_async_copy(
                        src_hbm.at[pl.ds(chunk * CHUNK, CHUNK)], sidx[buf],
                        semi[buf]).wait()
                    pltpu.make_async_copy(
                        dst_hbm.at[pl.ds(chunk * CHUNK, CHUNK)], didx[buf],
                        semi[buf]).wait()

                idx_load(s, 0)
                idx_wait(s, 0)
                pltpu.async_copy(h_hbm.at[cb].at[sidx[0]], rows[0], semg[0])

                @pl.when(s + NS < TCH)
                def _():
                    idx_load(s + NS, 1)

                def step(j, par):
                    cur = j * NS + s
                    nxt = cur + NS

                    @pl.when(nxt < TCH)
                    def _():
                        idx_wait(nxt, 1 - par)
                        pltpu.async_copy(h_hbm.at[cb].at[sidx[1 - par]],
                                         rows[1 - par], semg[1 - par])

                    @pl.when(cur < TCH)
                    def _():
                        pltpu.make_async_copy(h_hbm.at[cb].at[sidx[par]],
                                              rows[par], semg[par]).wait()
                        pltpu.sync_copy(rows[par], acc_sh.at[didx[par]],
                                        add=True)

                    @pl.when(nxt + NS < TCH)
                    def _():
                        idx_load(nxt + NS, par)

                def pair(t, _):
                    step(2 * t, 0)
                    step(2 * t + 1, 1)
                    return ()

                lax.fori_loop(0, NJ // 2, pair, ())
                step(NJ - 1, (NJ - 1) % 2)

        plsc.subcore_barrier()
        for cc in range(NCORES):
            cb = cc + NCORES * r

            @pl.when(c == cc)
            def _(cb=cb):
                pltpu.sync_copy(acc_sh.at[pl.ds(s * RPT, RPT), :],
                                agg_hbm.at[cb, pl.ds(s * RPT, RPT), :])

        if r + 1 < nblk // NCORES:
            plsc.subcore_barrier()


def _agg_call(h_blk, src, dst):
    nblk = h_blk.shape[0]
    zeros = jnp.zeros((RPT, LB), jnp.float32)
    mesh = plsc.VectorSubcoreMesh(core_axis_name="c", subcore_axis_name="s")
    fn = pl.kernel(
        functools.partial(_agg_body, nblk),
        out_type=jax.ShapeDtypeStruct((nblk, NPAD, LB), jnp.float32),
        mesh=mesh,
        scratch_types=[
            pltpu.VMEM((CHUNK,), jnp.int32),
            pltpu.VMEM((CHUNK,), jnp.int32),
            pltpu.VMEM((CHUNK,), jnp.int32),
            pltpu.VMEM((CHUNK,), jnp.int32),
            pltpu.VMEM((CHUNK, LB), jnp.float32),
            pltpu.VMEM((CHUNK, LB), jnp.float32),
            pltpu.VMEM_SHARED((NPAD, LB), jnp.float32),
            pltpu.SemaphoreType.DMA,
            pltpu.SemaphoreType.DMA,
            pltpu.SemaphoreType.DMA,
            pltpu.SemaphoreType.DMA,
        ],
    )
    return fn(h_blk, src, dst, zeros)


NB = 8
RB = NPAD // NB


def _prescale_body(x_ref, deg_ref, xs_ref, rout_ref, rin_ref):
    rout = lax.rsqrt(jnp.maximum(deg_ref[0, :, 0:1], 1.0))
    rin = lax.rsqrt(jnp.maximum(deg_ref[1, :, 0:1], 1.0))
    rout_ref[...] = rout
    rin_ref[...] = rin
    x = x_ref[...]
    for cb in range(D // LB):
        xs_ref[cb] = x[:, cb * LB:(cb + 1) * LB] * rout[:N]


def _prescale_call(x, deg):
    return pl.pallas_call(
        _prescale_body,
        out_shape=(
            jax.ShapeDtypeStruct((D // LB, N, LB), jnp.float32),
            jax.ShapeDtypeStruct((NPAD, 1), jnp.float32),
            jax.ShapeDtypeStruct((NPAD, 1), jnp.float32),
        ),
    )(x, deg)


def _mm_body(nblk, agg_ref, rin_ref, w_ref, z_ref, st_ref):
    i = pl.program_id(0)
    rin = rin_ref[...]
    z = jnp.zeros((RB, H), jnp.float32)
    for cb in range(nblk):
        a = agg_ref[cb] * rin
        z = z + jnp.dot(a, w_ref[cb * LB:(cb + 1) * LB, :],
                        preferred_element_type=jnp.float32)
    z_ref[...] = z
    cs = jnp.sum(z, axis=0, keepdims=True)
    css = jnp.sum(z * z, axis=0, keepdims=True)
    upd = jnp.concatenate([cs, css], axis=0)

    @pl.when(i == 0)
    def _():
        st_ref[...] = jnp.zeros((2, H), jnp.float32)

    st_ref[...] += upd


def _mm_call(agg, rin, w):
    nblk = agg.shape[0]
    return pl.pallas_call(
        functools.partial(_mm_body, nblk),
        grid=(NB,),
        in_specs=[
            pl.BlockSpec((nblk, RB, LB), lambda i: (0, i, 0)),
            pl.BlockSpec((RB, 1), lambda i: (i, 0)),
            pl.BlockSpec(w.shape, lambda i: (0, 0)),
        ],
        out_specs=(
            pl.BlockSpec((RB, H), lambda i: (i, 0)),
            pl.BlockSpec((2, H), lambda i: (0, 0)),
        ),
        out_shape=(
            jax.ShapeDtypeStruct((NPAD, H), jnp.float32),
            jax.ShapeDtypeStruct((2, H), jnp.float32),
        ),
    )(agg, rin, w)


def _norm_body(z_ref, st_ref, rout_ref, b_ref, g_ref, be_ref, h_ref):
    mu = st_ref[0:1, :] * (1.0 / N) + b_ref[...]
    ez2 = st_ref[1:2, :] * (1.0 / N)
    m0 = st_ref[0:1, :] * (1.0 / N)
    var = ez2 - m0 * m0
    zn = (z_ref[...] + b_ref[...] - mu) * lax.rsqrt(var + 1e-5) * g_ref[...] + be_ref[...]
    h = jnp.where(zn >= 0, zn, 0.01 * zn) * rout_ref[...]
    for co in range(H // LB):
        h_ref[co] = h[:, co * LB:(co + 1) * LB]


def _norm_call(z, st, rout, b, g, be):
    return pl.pallas_call(
        _norm_body,
        grid=(NB,),
        in_specs=[
            pl.BlockSpec((RB, H), lambda i: (i, 0)),
            pl.BlockSpec((2, H), lambda i: (0, 0)),
            pl.BlockSpec((RB, 1), lambda i: (i, 0)),
            pl.BlockSpec((1, H), lambda i: (0, 0)),
            pl.BlockSpec((1, H), lambda i: (0, 0)),
            pl.BlockSpec((1, H), lambda i: (0, 0)),
        ],
        out_specs=pl.BlockSpec((H // LB, RB, LB), lambda i: (0, i, 0)),
        out_shape=jax.ShapeDtypeStruct((H // LB, NPAD, LB), jnp.float32),
    )(z, st, rout, b.reshape(1, H), g.reshape(1, H), be.reshape(1, H))


def _readout_body(agg_ref, rin_ref, wg_ref, bg_ref, wc_ref, bc_ref, out_ref, acc):
    i = pl.program_id(0)
    rin = rin_ref[...]
    z = jnp.zeros((RB, H), jnp.float32)
    for cb in range(H // LB):
        a = agg_ref[cb] * rin
        z = z + jnp.dot(a, wg_ref[cb * LB:(cb + 1) * LB, :],
                        preferred_element_type=jnp.float32)
    z = z + bg_ref[...]
    hg = jnp.where(z >= 0, z, 0.01 * z)
    row = i * RB + lax.broadcasted_iota(jnp.int32, (RB, 1), 0)
    hg = jnp.where(row < N, hg, 0.0)

    @pl.when(i == 0)
    def _():
        acc[...] = jnp.zeros((1, H), jnp.float32)

    acc[...] += jnp.sum(hg, axis=0, keepdims=True)

    @pl.when(i == NB - 1)
    def _():
        m = acc[...] * (1.0 / N)
        out_ref[...] = jnp.dot(m, wc_ref[...],
                               preferred_element_type=jnp.float32) + bc_ref[...]


def _readout_call(agg, rin, wg, bg, wc, bc):
    return pl.pallas_call(
        _readout_body,
        grid=(NB,),
        in_specs=[
            pl.BlockSpec((H // LB, RB, LB), lambda i: (0, i, 0)),
            pl.BlockSpec((RB, 1), lambda i: (i, 0)),
            pl.BlockSpec((H, H), lambda i: (0, 0)),
            pl.BlockSpec((1, H), lambda i: (0, 0)),
            pl.BlockSpec((H, C), lambda i: (0, 0)),
            pl.BlockSpec((1, C), lambda i: (0, 0)),
        ],
        out_specs=pl.BlockSpec((1, C), lambda i: (0, 0)),
        out_shape=jax.ShapeDtypeStruct((1, C), jnp.float32),
        scratch_shapes=[pltpu.VMEM((1, H), jnp.float32)],
    )(agg, rin, wg, bg.reshape(1, H), wc, bc.reshape(1, C))


def kernel(node_feat, edge_index, edge_feat, W1, b1, g1, be1, W2, b2, g2, be2,
           We, bee, Wg, bg, Wc, bc):
    src = edge_index[0]
    dst = edge_index[1]
    deg = _deg_call(src, dst)
    xs, rout, rin = _prescale_call(node_feat, deg)
    agg1 = _agg_call(xs, src, dst)
    z1, st1 = _mm_call(agg1, rin, W1)
    h1 = _norm_call(z1, st1, rout, b1, g1, be1)
    agg2 = _agg_call(h1, src, dst)
    z2, st2 = _mm_call(agg2, rin, W2)
    h2 = _norm_call(z2, st2, rout, b2, g2, be2)
    agg3 = _agg_call(h2, src, dst)
    return _readout_call(agg3, rin, Wg[:H], bg, Wc, bc)
